# baseline (speedup 1.0000x reference)
import functools

import jax
import jax.numpy as jnp
from jax import lax
from jax.experimental import pallas as pl
from jax.experimental.pallas import tpu as pltpu


def _body(n, h, bi, n_blk,
          hs_ref, skipT_ref, ip3_ref, ip_col_ref, bd_ref, hp_ref, ti_ref,
          fi_ref, out_ip_ref, out_h_ref, acc_sc, acc_skip, diag_col):
    b = pl.program_id(0)
    gbase = b * bi

    ip_blk = ip3_ref[0]
    skipT = skipT_ref[0]
    jrow = lax.broadcasted_iota(jnp.int32, (n, bi), 0)
    icol = lax.broadcasted_iota(jnp.int32, (n, bi), 1) + gbase
    isdiag = jrow == icol
    w = jnp.where(isdiag, 0.0, skipT) * ip_blk

    @pl.when(b == 0)
    def _init():
        acc_sc[...] = jnp.zeros_like(acc_sc)
        acc_skip[...] = jnp.zeros_like(acc_skip)
        diag_col[...] = jnp.zeros_like(diag_col)

    diag_col[...] += jnp.sum(jnp.where(isdiag, skipT, 0.0), axis=1,
                             keepdims=True)
    acc_skip[...] += jnp.sum(w, axis=1, keepdims=True)

    acc = acc_sc[...]
    for i in range(bi):
        acc = acc + hs_ref[i] * w[:, i:i + 1]
    acc_sc[...] = acc

    @pl.when(b == n_blk - 1)
    def _fin():
        ip_col = ip_col_ref[...]
        dcol = diag_col[...]
        p_t = bd_ref[:, 0:1]
        p_f = bd_ref[:, 1:2]
        pbt = ip_col * dcol * p_t
        pbf = ip_col * dcol * p_f
        jj = lax.broadcasted_iota(jnp.int32, (n, n), 1)
        m = (jnp.where(ti_ref[...] == jj, pbt, 0.0)
             + jnp.where(fi_ref[...] == jj, pbf, 0.0))
        g = jnp.concatenate(
            [hp_ref[...], jnp.ones((n, 1), jnp.float32)], axis=1)
        seg = lax.dot_general(m, g, (((0,), (0,)), ((), ())),
                              preferred_element_type=jnp.float32)
        new_ip = seg[:, h:h + 1] + acc_skip[...]
        out_ip_ref[...] = new_ip
        out_h_ref[...] = (seg[:, :h] + acc_sc[...]) / (new_ip + 1e-7)


def kernel(step, instruction_pointer, hidden_states, hidden_state_proposals,
           hidden_state_skip_proposals, skip_decisions, branch_decisions,
           node_embeddings, true_indexes, false_indexes):
    n, h = hidden_state_proposals.shape
    bi = 32 if n % 32 == 0 else n
    n_blk = n // bi

    skip_t3 = skip_decisions.reshape(n_blk, bi, n).transpose(0, 2, 1)
    ip3 = instruction_pointer.reshape(n_blk, 1, bi)
    ip_col = instruction_pointer.reshape(n, 1)
    ti = true_indexes.reshape(n, 1)
    fi = false_indexes.reshape(n, 1)

    grid = (n_blk,)
    out_ip, out_h = pl.pallas_call(
        functools.partial(_body, n, h, bi, n_blk),
        grid=grid,
        in_specs=[
            pl.BlockSpec((bi, n, h), lambda b: (b, 0, 0)),
            pl.BlockSpec((1, n, bi), lambda b: (b, 0, 0)),
            pl.BlockSpec((1, 1, bi), lambda b: (b, 0, 0)),
            pl.BlockSpec((n, 1), lambda b: (0, 0)),
            pl.BlockSpec((n, 2), lambda b: (0, 0)),
            pl.BlockSpec((n, h), lambda b: (0, 0)),
            pl.BlockSpec((n, 1), lambda b: (0, 0)),
            pl.BlockSpec((n, 1), lambda b: (0, 0)),
        ],
        out_specs=[
            pl.BlockSpec((n, 1), lambda b: (0, 0)),
            pl.BlockSpec((n, h), lambda b: (0, 0)),
        ],
        out_shape=[
            jax.ShapeDtypeStruct((n, 1), jnp.float32),
            jax.ShapeDtypeStruct((n, h), jnp.float32),
        ],
        scratch_shapes=[
            pltpu.VMEM((n, h), jnp.float32),
            pltpu.VMEM((n, 1), jnp.float32),
            pltpu.VMEM((n, 1), jnp.float32),
        ],
    )(hidden_state_skip_proposals, skip_t3, ip3, ip_col, branch_decisions,
      hidden_state_proposals, ti, fi)
    return out_ip.reshape(n), out_h
